# baseline (speedup 1.0000x reference)
import functools

import jax
import jax.numpy as jnp
from jax import lax
from jax.experimental import pallas as pl
from jax.experimental.pallas import tpu as pltpu
from jax.experimental.pallas import tpu_sc as plsc

F32 = jnp.float32

_N = 10000
_E = 320000
_D_IN = 128
_HID = 256
_HALF = 128
_OUT = 128
_L = 8
_CHUNK = 64

_NTILE = 16
_NPAD = 10240
_RPT = 640
_EP = 327680
_IDXROWS = _EP // 128
_ROWS_PER_TILE = _IDXROWS // _NTILE

_IG = 16

_BN = 400
_GRID = _N // _BN

def _mesh():
    return plsc.VectorSubcoreMesh(core_axis_name="c", subcore_axis_name="s",
                                  num_cores=2, num_subcores=_NTILE)


def _sc_agg_body(h2_hbm, src_hbm, dst_hbm, out_hbm, srcI, dstI, rows0, rows1,
                 acc, semg0, semg1, sems0, sems1):
    c = lax.axis_index("c")
    s = lax.axis_index("s")
    zero = jnp.zeros((16,), F32)

    @pl.loop(0, 128)
    def _z(i):
        for j in range(8):
            rows0[i, pl.ds(j * 16, 16)] = zero

    rbase = s * _RPT
    for j in range(_RPT // 128):
        pltpu.sync_copy(rows0, acc.at[pl.ds(rbase + j * 128, 128)])

    plsc.subcore_barrier()

    ib = s * _ROWS_PER_TILE
    bufs = (rows0, rows1)
    gsems = (semg0, semg1)
    ssems = (sems0, sems1)

    @pl.loop(0, _ROWS_PER_TILE // _IG)
    def _g(g):
        pltpu.sync_copy(src_hbm.at[c, pl.ds(ib + g * _IG, _IG)], srcI)
        pltpu.sync_copy(dst_hbm.at[pl.ds(ib + g * _IG, _IG)], dstI)

        scat = [None, None]
        gh = pltpu.async_copy(h2_hbm.at[srcI.at[0]], bufs[0], gsems[0])
        for i in range(_IG):
            b = i % 2
            gh.wait()
            if i < _IG - 1:
                if scat[1 - b] is not None:
                    scat[1 - b].wait()
                gh = pltpu.async_copy(h2_hbm.at[srcI.at[i + 1]],
                                      bufs[1 - b], gsems[1 - b])
            scat[b] = pltpu.async_copy(bufs[b], acc.at[dstI.at[i]],
                                       ssems[b], add=True)
        scat[0].wait()
        scat[1].wait()

    plsc.subcore_barrier()
    pltpu.sync_copy(acc.at[pl.ds(rbase, _RPT)], out_hbm.at[c, pl.ds(rbase, _RPT)])


@jax.jit
def _sc_agg(h2, src3, dst3):
    run = pl.kernel(
        _sc_agg_body,
        out_type=jax.ShapeDtypeStruct((2, _NPAD, _HALF), F32),
        mesh=_mesh(),
        scratch_types=[
            pltpu.VMEM((_IG, 128), jnp.int32),
            pltpu.VMEM((_IG, 128), jnp.int32),
            pltpu.VMEM((128, _HALF), F32),
            pltpu.VMEM((128, _HALF), F32),
            pltpu.VMEM_SHARED((_NPAD, _HALF), F32),
            pltpu.SemaphoreType.DMA,
            pltpu.SemaphoreType.DMA,
            pltpu.SemaphoreType.DMA,
            pltpu.SemaphoreType.DMA,
        ],
    )
    return run(h2, src3, dst3)


def _sc_deg_body(dst_hbm, out_hbm, dstI, ones_rows, acc):
    c = lax.axis_index("c")
    s = lax.axis_index("s")
    zero = jnp.zeros((16,), F32)
    one = jnp.ones((16,), F32)

    @pl.loop(0, 128)
    def _z(i):
        for j in range(8):
            ones_rows[i, pl.ds(j * 16, 16)] = zero

    rbase = s * _RPT
    for j in range(_RPT // 128):
        pltpu.sync_copy(ones_rows, acc.at[pl.ds(rbase + j * 128, 128)])

    @pl.loop(0, 128)
    def _f(i):
        for j in range(8):
            ones_rows[i, pl.ds(j * 16, 16)] = one

    hrows = _ROWS_PER_TILE // 2
    ib = c * (_IDXROWS // 2) + s * hrows
    plsc.subcore_barrier()

    @pl.loop(0, hrows // _IG)
    def _g(g):
        pltpu.sync_copy(dst_hbm.at[pl.ds(ib + g * _IG, _IG)], dstI)
        for i in range(_IG):
            pltpu.sync_copy(ones_rows, acc.at[dstI.at[i]], add=True)

    plsc.subcore_barrier()
    pltpu.sync_copy(acc.at[pl.ds(rbase, _RPT)], out_hbm.at[c, pl.ds(rbase, _RPT)])


@jax.jit
def _sc_deg(dst3):
    run = pl.kernel(
        _sc_deg_body,
        out_type=jax.ShapeDtypeStruct((2, _NPAD, 128), F32),
        mesh=_mesh(),
        scratch_types=[
            pltpu.VMEM((_IG, 128), jnp.int32),
            pltpu.VMEM((128, 128), F32),
            pltpu.VMEM_SHARED((_NPAD, 128), F32),
        ],
    )
    return run(dst3)


_DOT = functools.partial(jnp.dot, preferred_element_type=F32,
                         precision=lax.Precision.HIGHEST)


def _tc_in_body(x_ref, w_ref, b_ref, hc_ref):
    h = jnp.maximum(_DOT(x_ref[...], w_ref[...]) + b_ref[...], 0.0)
    hc_ref[0] = h[:, :_HALF]
    hc_ref[1] = h[:, _HALF:]


def _tc_layer_body(hc_ref, msg_ref, dinv_ref, sig_ref, w_ref, b_ref, g_ref,
                   bb_ref, hco_ref, sigo_ref):
    h = jnp.concatenate([hc_ref[0], hc_ref[1]], axis=1)
    mr = jnp.concatenate([msg_ref[0], msg_ref[1]], axis=1)
    dinv = 1.0 / (dinv_ref[0, :, :1] + dinv_ref[1, :, :1] + 1.0)
    m = (mr + h) * dinv
    w = w_ref[...]
    logits = _DOT(h, w[:_HID]) + _DOT(m, w[_HID:]) + b_ref[...]
    z = logits - jnp.max(logits, axis=1, keepdims=True)
    e = jnp.exp(z)
    p = e / jnp.sum(e, axis=1, keepdims=True)
    r64 = lax.broadcasted_iota(jnp.int32, (_CHUNK, _CHUNK), 0)
    c64 = lax.broadcasted_iota(jnp.int32, (_CHUNK, _CHUNK), 1)
    tril = (r64 <= c64).astype(F32)
    cum = _DOT(p, tril)
    sig = sig_ref[...]
    raw = sig + (1.0 - sig) * cum
    r2 = lax.broadcasted_iota(jnp.int32, (_CHUNK, _HID), 0)
    c2 = lax.broadcasted_iota(jnp.int32, (_CHUNK, _HID), 1)
    repm = (r2 == c2 // (_HID // _CHUNK)).astype(F32)
    sigf = _DOT(raw, repm)
    out = h * sigf + m * (1.0 - sigf)
    mu = jnp.mean(out, axis=1, keepdims=True)
    var = jnp.mean((out - mu) ** 2, axis=1, keepdims=True)
    hn = (out - mu) * lax.rsqrt(var + 1e-5) * g_ref[...] + bb_ref[...]
    hco_ref[0] = hn[:, :_HALF]
    hco_ref[1] = hn[:, _HALF:]
    sigo_ref[...] = raw


def _tc_out_body(hc_ref, w_ref, b_ref, o_ref):
    h = jnp.concatenate([hc_ref[0], hc_ref[1]], axis=1)
    o_ref[...] = _DOT(h, w_ref[...]) + b_ref[...]


def _tc_input(x, W_in, b_in):
    return pl.pallas_call(
        _tc_in_body,
        grid=(_GRID,),
        in_specs=[
            pl.BlockSpec((_BN, _D_IN), lambda i: (i, 0)),
            pl.BlockSpec((_D_IN, _HID), lambda i: (0, 0)),
            pl.BlockSpec((1, _HID), lambda i: (0, 0)),
        ],
        out_specs=pl.BlockSpec((2, _BN, _HALF), lambda i: (0, i, 0)),
        out_shape=jax.ShapeDtypeStruct((2, _N, _HALF), F32),
    )(x, W_in, b_in)


def _tc_layer(hc, msg, dinv, sig, wl, bl, gl, bbl):
    return pl.pallas_call(
        _tc_layer_body,
        grid=(_GRID,),
        in_specs=[
            pl.BlockSpec((2, _BN, _HALF), lambda i: (0, i, 0)),
            pl.BlockSpec((2, _BN, _HALF), lambda i: (0, i, 0)),
            pl.BlockSpec((2, _BN, 128), lambda i: (0, i, 0)),
            pl.BlockSpec((_BN, _CHUNK), lambda i: (i, 0)),
            pl.BlockSpec((2 * _HID, _CHUNK), lambda i: (0, 0)),
            pl.BlockSpec((1, _CHUNK), lambda i: (0, 0)),
            pl.BlockSpec((1, _HID), lambda i: (0, 0)),
            pl.BlockSpec((1, _HID), lambda i: (0, 0)),
        ],
        out_specs=[
            pl.BlockSpec((2, _BN, _HALF), lambda i: (0, i, 0)),
            pl.BlockSpec((_BN, _CHUNK), lambda i: (i, 0)),
        ],
        out_shape=[
            jax.ShapeDtypeStruct((2, _N, _HALF), F32),
            jax.ShapeDtypeStruct((_N, _CHUNK), F32),
        ],
    )(hc, msg, dinv, sig, wl, bl, gl, bbl)


def _tc_output(hc, W_out, b_out):
    return pl.pallas_call(
        _tc_out_body,
        grid=(_GRID,),
        in_specs=[
            pl.BlockSpec((2, _BN, _HALF), lambda i: (0, i, 0)),
            pl.BlockSpec((_HID, _OUT), lambda i: (0, 0)),
            pl.BlockSpec((1, _OUT), lambda i: (0, 0)),
        ],
        out_specs=pl.BlockSpec((_BN, _OUT), lambda i: (i, 0)),
        out_shape=jax.ShapeDtypeStruct((_N, _OUT), F32),
    )(hc, W_out, b_out)


def kernel(x, edge_index, W_in, b_in, tm_W, tm_b, ln_g, ln_b, W_out, b_out):
    pad = _EP - _E
    srcp = jnp.concatenate([edge_index[0], jnp.zeros((pad,), jnp.int32)])
    dstp = jnp.concatenate([edge_index[1], jnp.full((pad,), _N, jnp.int32)])
    src3 = jnp.stack([srcp, srcp + _N]).reshape(2, _IDXROWS, 128)
    dst3 = dstp.reshape(_IDXROWS, 128)

    hc = _tc_input(x, W_in, b_in.reshape(1, _HID))
    dinv = _sc_deg(dst3)
    sig = jnp.zeros((_N, _CHUNK), F32)
    for l in range(_L):
        msg = _sc_agg(hc.reshape(2 * _N, _HALF), src3, dst3)
        hc, sig = _tc_layer(hc, msg, dinv, sig, tm_W[l],
                            tm_b[l].reshape(1, _CHUNK),
                            ln_g[l].reshape(1, _HID), ln_b[l].reshape(1, _HID))
    return _tc_output(hc, W_out, b_out.reshape(1, _OUT))


# speedup vs baseline: 1.0349x; 1.0349x over previous
import functools

import jax
import jax.numpy as jnp
from jax import lax
from jax.experimental import pallas as pl
from jax.experimental.pallas import tpu as pltpu
from jax.experimental.pallas import tpu_sc as plsc

F32 = jnp.float32

_N = 10000
_E = 320000
_D_IN = 128
_HID = 256
_HALF = 128
_OUT = 128
_L = 8
_CHUNK = 64

_NTILE = 16
_NPAD = 10240
_RPT = 640
_EP = 327680
_IDXROWS = _EP // 128
_ROWS_PER_TILE = _IDXROWS // _NTILE

_IG = 16

_BN = 400
_GRID = _N // _BN

def _mesh():
    return plsc.VectorSubcoreMesh(core_axis_name="c", subcore_axis_name="s",
                                  num_cores=2, num_subcores=_NTILE)


def _sc_agg_body(h2_hbm, src_hbm, dst_hbm, out_hbm, srcI, dstI, rows0, rows1,
                 acc, semg0, semg1, sems0, sems1):
    c = lax.axis_index("c")
    s = lax.axis_index("s")
    zero = jnp.zeros((16,), F32)

    @pl.loop(0, 128)
    def _z(i):
        for j in range(8):
            rows0[i, pl.ds(j * 16, 16)] = zero

    rbase = s * _RPT
    for j in range(_RPT // 128):
        pltpu.sync_copy(rows0, acc.at[pl.ds(rbase + j * 128, 128)])

    plsc.subcore_barrier()

    off = c * _N
    ib = s * _ROWS_PER_TILE
    bufs = (rows0, rows1)
    gsems = (semg0, semg1)
    ssems = (sems0, sems1)

    @pl.loop(0, _ROWS_PER_TILE // _IG)
    def _g(g):
        pltpu.sync_copy(src_hbm.at[pl.ds(ib + g * _IG, _IG)], srcI)
        pltpu.sync_copy(dst_hbm.at[pl.ds(ib + g * _IG, _IG)], dstI)

        @pl.loop(0, _IG)
        def _o(i):
            for j in range(8):
                srcI[i, pl.ds(j * 16, 16)] = srcI[i, pl.ds(j * 16, 16)] + off

        scat = [None, None]
        gh = pltpu.async_copy(h2_hbm.at[srcI.at[0]], bufs[0], gsems[0])
        for i in range(_IG):
            b = i % 2
            gh.wait()
            if i < _IG - 1:
                if scat[1 - b] is not None:
                    scat[1 - b].wait()
                gh = pltpu.async_copy(h2_hbm.at[srcI.at[i + 1]],
                                      bufs[1 - b], gsems[1 - b])
            scat[b] = pltpu.async_copy(bufs[b], acc.at[dstI.at[i]],
                                       ssems[b], add=True)
        scat[0].wait()
        scat[1].wait()

    plsc.subcore_barrier()
    pltpu.sync_copy(acc.at[pl.ds(rbase, _RPT)], out_hbm.at[c, pl.ds(rbase, _RPT)])


@jax.jit
def _sc_agg(h2, src3, dst3):
    run = pl.kernel(
        _sc_agg_body,
        out_type=jax.ShapeDtypeStruct((2, _NPAD, _HALF), F32),
        mesh=_mesh(),
        scratch_types=[
            pltpu.VMEM((_IG, 128), jnp.int32),
            pltpu.VMEM((_IG, 128), jnp.int32),
            pltpu.VMEM((128, _HALF), F32),
            pltpu.VMEM((128, _HALF), F32),
            pltpu.VMEM_SHARED((_NPAD, _HALF), F32),
            pltpu.SemaphoreType.DMA,
            pltpu.SemaphoreType.DMA,
            pltpu.SemaphoreType.DMA,
            pltpu.SemaphoreType.DMA,
        ],
    )
    return run(h2, src3, dst3)


def _sc_deg_body(dst_hbm, out_hbm, dstI, ones_rows, acc):
    c = lax.axis_index("c")
    s = lax.axis_index("s")
    zero = jnp.zeros((16,), F32)
    one = jnp.ones((16,), F32)

    @pl.loop(0, 128)
    def _z(i):
        for j in range(8):
            ones_rows[i, pl.ds(j * 16, 16)] = zero

    rbase = s * _RPT
    for j in range(_RPT // 128):
        pltpu.sync_copy(ones_rows, acc.at[pl.ds(rbase + j * 128, 128)])

    @pl.loop(0, 128)
    def _f(i):
        for j in range(8):
            ones_rows[i, pl.ds(j * 16, 16)] = one

    hrows = _ROWS_PER_TILE // 2
    ib = c * (_IDXROWS // 2) + s * hrows
    plsc.subcore_barrier()

    @pl.loop(0, hrows // _IG)
    def _g(g):
        pltpu.sync_copy(dst_hbm.at[pl.ds(ib + g * _IG, _IG)], dstI)
        for i in range(_IG):
            pltpu.sync_copy(ones_rows, acc.at[dstI.at[i]], add=True)

    plsc.subcore_barrier()
    pltpu.sync_copy(acc.at[pl.ds(rbase, _RPT)], out_hbm.at[c, pl.ds(rbase, _RPT)])


@jax.jit
def _sc_deg(dst3):
    run = pl.kernel(
        _sc_deg_body,
        out_type=jax.ShapeDtypeStruct((2, _NPAD, 128), F32),
        mesh=_mesh(),
        scratch_types=[
            pltpu.VMEM((_IG, 128), jnp.int32),
            pltpu.VMEM((128, 128), F32),
            pltpu.VMEM_SHARED((_NPAD, 128), F32),
        ],
    )
    return run(dst3)


_DOT = functools.partial(jnp.dot, preferred_element_type=F32,
                         precision=lax.Precision.HIGHEST)


def _tc_in_body(x_ref, w_ref, b_ref, hc_ref):
    h = jnp.maximum(_DOT(x_ref[...], w_ref[...]) + b_ref[...], 0.0)
    hc_ref[0] = h[:, :_HALF]
    hc_ref[1] = h[:, _HALF:]


def _tc_layer_body(hc_ref, msg_ref, dinv_ref, sig_ref, w_ref, b_ref, g_ref,
                   bb_ref, hco_ref, sigo_ref):
    h = jnp.concatenate([hc_ref[0], hc_ref[1]], axis=1)
    mr = jnp.concatenate([msg_ref[0], msg_ref[1]], axis=1)
    dinv = 1.0 / (dinv_ref[0, :, :1] + dinv_ref[1, :, :1] + 1.0)
    m = (mr + h) * dinv
    w = w_ref[...]
    logits = _DOT(h, w[:_HID]) + _DOT(m, w[_HID:]) + b_ref[...]
    z = logits - jnp.max(logits, axis=1, keepdims=True)
    e = jnp.exp(z)
    p = e / jnp.sum(e, axis=1, keepdims=True)
    r64 = lax.broadcasted_iota(jnp.int32, (_CHUNK, _CHUNK), 0)
    c64 = lax.broadcasted_iota(jnp.int32, (_CHUNK, _CHUNK), 1)
    tril = (r64 <= c64).astype(F32)
    cum = _DOT(p, tril)
    sig = sig_ref[...]
    raw = sig + (1.0 - sig) * cum
    r2 = lax.broadcasted_iota(jnp.int32, (_CHUNK, _HID), 0)
    c2 = lax.broadcasted_iota(jnp.int32, (_CHUNK, _HID), 1)
    repm = (r2 == c2 // (_HID // _CHUNK)).astype(F32)
    sigf = _DOT(raw, repm)
    out = h * sigf + m * (1.0 - sigf)
    mu = jnp.mean(out, axis=1, keepdims=True)
    var = jnp.mean((out - mu) ** 2, axis=1, keepdims=True)
    hn = (out - mu) * lax.rsqrt(var + 1e-5) * g_ref[...] + bb_ref[...]
    hco_ref[0] = hn[:, :_HALF]
    hco_ref[1] = hn[:, _HALF:]
    sigo_ref[...] = raw


def _tc_out_body(hc_ref, w_ref, b_ref, o_ref):
    h = jnp.concatenate([hc_ref[0], hc_ref[1]], axis=1)
    o_ref[...] = _DOT(h, w_ref[...]) + b_ref[...]


def _tc_input(x, W_in, b_in):
    return pl.pallas_call(
        _tc_in_body,
        grid=(_GRID,),
        in_specs=[
            pl.BlockSpec((_BN, _D_IN), lambda i: (i, 0)),
            pl.BlockSpec((_D_IN, _HID), lambda i: (0, 0)),
            pl.BlockSpec((1, _HID), lambda i: (0, 0)),
        ],
        out_specs=pl.BlockSpec((2, _BN, _HALF), lambda i: (0, i, 0)),
        out_shape=jax.ShapeDtypeStruct((2, _N, _HALF), F32),
    )(x, W_in, b_in)


def _tc_layer(hc, msg, dinv, sig, wl, bl, gl, bbl):
    return pl.pallas_call(
        _tc_layer_body,
        grid=(_GRID,),
        in_specs=[
            pl.BlockSpec((2, _BN, _HALF), lambda i: (0, i, 0)),
            pl.BlockSpec((2, _BN, _HALF), lambda i: (0, i, 0)),
            pl.BlockSpec((2, _BN, 128), lambda i: (0, i, 0)),
            pl.BlockSpec((_BN, _CHUNK), lambda i: (i, 0)),
            pl.BlockSpec((2 * _HID, _CHUNK), lambda i: (0, 0)),
            pl.BlockSpec((1, _CHUNK), lambda i: (0, 0)),
            pl.BlockSpec((1, _HID), lambda i: (0, 0)),
            pl.BlockSpec((1, _HID), lambda i: (0, 0)),
        ],
        out_specs=[
            pl.BlockSpec((2, _BN, _HALF), lambda i: (0, i, 0)),
            pl.BlockSpec((_BN, _CHUNK), lambda i: (i, 0)),
        ],
        out_shape=[
            jax.ShapeDtypeStruct((2, _N, _HALF), F32),
            jax.ShapeDtypeStruct((_N, _CHUNK), F32),
        ],
    )(hc, msg, dinv, sig, wl, bl, gl, bbl)


def _tc_output(hc, W_out, b_out):
    return pl.pallas_call(
        _tc_out_body,
        grid=(_GRID,),
        in_specs=[
            pl.BlockSpec((2, _BN, _HALF), lambda i: (0, i, 0)),
            pl.BlockSpec((_HID, _OUT), lambda i: (0, 0)),
            pl.BlockSpec((1, _OUT), lambda i: (0, 0)),
        ],
        out_specs=pl.BlockSpec((_BN, _OUT), lambda i: (i, 0)),
        out_shape=jax.ShapeDtypeStruct((_N, _OUT), F32),
    )(hc, W_out, b_out)


def kernel(x, edge_index, W_in, b_in, tm_W, tm_b, ln_g, ln_b, W_out, b_out):
    pad = _EP - _E
    srcp = jnp.concatenate([edge_index[0], jnp.zeros((pad,), jnp.int32)])
    dstp = jnp.concatenate([edge_index[1], jnp.full((pad,), _N, jnp.int32)])
    src3 = srcp.reshape(_IDXROWS, 128)
    dst3 = dstp.reshape(_IDXROWS, 128)

    hc = _tc_input(x, W_in, b_in.reshape(1, _HID))
    dinv = _sc_deg(dst3)
    sig = jnp.zeros((_N, _CHUNK), F32)
    for l in range(_L):
        msg = _sc_agg(hc.reshape(2 * _N, _HALF), src3, dst3)
        hc, sig = _tc_layer(hc, msg, dinv, sig, tm_W[l],
                            tm_b[l].reshape(1, _CHUNK),
                            ln_g[l].reshape(1, _HID), ln_b[l].reshape(1, _HID))
    return _tc_output(hc, W_out, b_out.reshape(1, _OUT))


# speedup vs baseline: 1.0599x; 1.0241x over previous
import functools

import jax
import jax.numpy as jnp
from jax import lax
from jax.experimental import pallas as pl
from jax.experimental.pallas import tpu as pltpu
from jax.experimental.pallas import tpu_sc as plsc

F32 = jnp.float32

_N = 10000
_E = 320000
_D_IN = 128
_HID = 256
_HALF = 128
_OUT = 128
_L = 8
_CHUNK = 64

_NTILE = 16
_NPAD = 10240
_RPT = 640
_EP = 327680
_IDXROWS = _EP // 128
_ROWS_PER_TILE = _IDXROWS // _NTILE

_IG = 16

_BN = 400
_GRID = _N // _BN

def _mesh():
    return plsc.VectorSubcoreMesh(core_axis_name="c", subcore_axis_name="s",
                                  num_cores=2, num_subcores=_NTILE)


def _sc_agg_body(h2_hbm, src_hbm, dst_hbm, out_hbm, srcI, dstI, rows0, rows1,
                 acc, semg0, semg1, sems0, sems1):
    c = lax.axis_index("c")
    s = lax.axis_index("s")
    zero = jnp.zeros((16,), F32)

    @pl.loop(0, 128)
    def _z(i):
        for j in range(8):
            rows0[i, pl.ds(j * 16, 16)] = zero

    rbase = s * _RPT
    for j in range(_RPT // 128):
        pltpu.sync_copy(rows0, acc.at[pl.ds(rbase + j * 128, 128)])

    plsc.subcore_barrier()

    off = c * _N
    ib = s * _ROWS_PER_TILE
    bufs = (rows0, rows1)
    gsems = (semg0, semg1)
    ssems = (sems0, sems1)

    @pl.loop(0, _ROWS_PER_TILE // _IG)
    def _g(g):
        pltpu.sync_copy(src_hbm.at[pl.ds(ib + g * _IG, _IG)], srcI)
        pltpu.sync_copy(dst_hbm.at[pl.ds(ib + g * _IG, _IG)], dstI)

        @pl.loop(0, _IG)
        def _o(i):
            for j in range(8):
                srcI[i, pl.ds(j * 16, 16)] = srcI[i, pl.ds(j * 16, 16)] + off

        scat = [None, None]
        gh = pltpu.async_copy(h2_hbm.at[srcI.at[0]], bufs[0], gsems[0])
        for i in range(_IG):
            b = i % 2
            gh.wait()
            if i < _IG - 1:
                if scat[1 - b] is not None:
                    scat[1 - b].wait()
                gh = pltpu.async_copy(h2_hbm.at[srcI.at[i + 1]],
                                      bufs[1 - b], gsems[1 - b])
            scat[b] = pltpu.async_copy(bufs[b], acc.at[dstI.at[i]],
                                       ssems[b], add=True)
        scat[0].wait()
        scat[1].wait()

    plsc.subcore_barrier()
    pltpu.sync_copy(acc.at[pl.ds(rbase, _RPT)], out_hbm.at[c, pl.ds(rbase, _RPT)])


@jax.jit
def _sc_agg(h2, src3, dst3):
    run = pl.kernel(
        _sc_agg_body,
        out_type=jax.ShapeDtypeStruct((2, _NPAD, _HALF), F32),
        mesh=_mesh(),
        scratch_types=[
            pltpu.VMEM((_IG, 128), jnp.int32),
            pltpu.VMEM((_IG, 128), jnp.int32),
            pltpu.VMEM((128, _HALF), F32),
            pltpu.VMEM((128, _HALF), F32),
            pltpu.VMEM_SHARED((_NPAD, _HALF), F32),
            pltpu.SemaphoreType.DMA,
            pltpu.SemaphoreType.DMA,
            pltpu.SemaphoreType.DMA,
            pltpu.SemaphoreType.DMA,
        ],
    )
    return run(h2, src3, dst3)


def _sc_deg_body(dst_hbm, out_hbm, dstI, ones_rows, dloc, acc):
    c = lax.axis_index("c")
    s = lax.axis_index("s")
    zero = jnp.zeros((16,), F32)
    one = jnp.ones((16,), F32)

    @pl.loop(0, 128)
    def _z(i):
        for j in range(8):
            ones_rows[i, pl.ds(j * 16, 16)] = zero

    rbase = s * _RPT
    for j in range(_RPT // 128):
        pltpu.sync_copy(ones_rows, acc.at[pl.ds(rbase + j * 128, 128)])

    @pl.loop(0, 128)
    def _f(i):
        for j in range(8):
            ones_rows[i, pl.ds(j * 16, 16)] = one

    ib = s * _ROWS_PER_TILE
    plsc.subcore_barrier()

    @pl.loop(0, _ROWS_PER_TILE // _IG)
    def _g(g):
        pltpu.sync_copy(dst_hbm.at[pl.ds(ib + g * _IG, _IG)], dstI)
        for i in range(_IG):
            pltpu.sync_copy(ones_rows, acc.at[dstI.at[i]], add=True)

    plsc.subcore_barrier()

    @pl.when(c == 0)
    def _dump():
        @pl.loop(0, _RPT // 128)
        def _p(q):
            pltpu.sync_copy(acc.at[pl.ds(rbase + q * 128, 128)], dloc)

            @pl.loop(0, 128)
            def _d(r):
                for j in range(8):
                    dloc[r, pl.ds(j * 16, 16)] = 1.0 / (dloc[r, pl.ds(j * 16, 16)] + 1.0)

            pltpu.sync_copy(dloc, out_hbm.at[pl.ds(rbase + q * 128, 128)])


@jax.jit
def _sc_deg(dst3):
    run = pl.kernel(
        _sc_deg_body,
        out_type=jax.ShapeDtypeStruct((_NPAD, 128), F32),
        mesh=_mesh(),
        scratch_types=[
            pltpu.VMEM((_IG, 128), jnp.int32),
            pltpu.VMEM((128, 128), F32),
            pltpu.VMEM((128, 128), F32),
            pltpu.VMEM_SHARED((_NPAD, 128), F32),
        ],
    )
    return run(dst3)


_DOT = functools.partial(jnp.dot, preferred_element_type=F32,
                         precision=lax.Precision.HIGHEST)


def _tc_in_body(x_ref, w_ref, b_ref, hc_ref):
    h = jnp.maximum(_DOT(x_ref[...], w_ref[...]) + b_ref[...], 0.0)
    hc_ref[0] = h[:, :_HALF]
    hc_ref[1] = h[:, _HALF:]


def _tc_layer_body(hc_ref, msg_ref, dinv_ref, sig_ref, w_ref, b_ref, g_ref,
                   bb_ref, hco_ref, sigo_ref):
    h = jnp.concatenate([hc_ref[0], hc_ref[1]], axis=1)
    mr = jnp.concatenate([msg_ref[0], msg_ref[1]], axis=1)
    dinv = dinv_ref[...][:, :1]
    m = (mr + h) * dinv
    w = w_ref[...]
    logits = _DOT(h, w[:_HID]) + _DOT(m, w[_HID:]) + b_ref[...]
    z = logits - jnp.max(logits, axis=1, keepdims=True)
    e = jnp.exp(z)
    p = e / jnp.sum(e, axis=1, keepdims=True)
    r64 = lax.broadcasted_iota(jnp.int32, (_CHUNK, _CHUNK), 0)
    c64 = lax.broadcasted_iota(jnp.int32, (_CHUNK, _CHUNK), 1)
    tril = (r64 <= c64).astype(F32)
    cum = _DOT(p, tril)
    sig = sig_ref[...]
    raw = sig + (1.0 - sig) * cum
    r2 = lax.broadcasted_iota(jnp.int32, (_CHUNK, _HID), 0)
    c2 = lax.broadcasted_iota(jnp.int32, (_CHUNK, _HID), 1)
    repm = (r2 == c2 // (_HID // _CHUNK)).astype(F32)
    sigf = _DOT(raw, repm)
    out = h * sigf + m * (1.0 - sigf)
    mu = jnp.mean(out, axis=1, keepdims=True)
    var = jnp.mean((out - mu) ** 2, axis=1, keepdims=True)
    hn = (out - mu) * lax.rsqrt(var + 1e-5) * g_ref[...] + bb_ref[...]
    hco_ref[0] = hn[:, :_HALF]
    hco_ref[1] = hn[:, _HALF:]
    sigo_ref[...] = raw


def _tc_out_body(hc_ref, w_ref, b_ref, o_ref):
    h = jnp.concatenate([hc_ref[0], hc_ref[1]], axis=1)
    o_ref[...] = _DOT(h, w_ref[...]) + b_ref[...]


def _tc_input(x, W_in, b_in):
    return pl.pallas_call(
        _tc_in_body,
        grid=(_GRID,),
        in_specs=[
            pl.BlockSpec((_BN, _D_IN), lambda i: (i, 0)),
            pl.BlockSpec((_D_IN, _HID), lambda i: (0, 0)),
            pl.BlockSpec((1, _HID), lambda i: (0, 0)),
        ],
        out_specs=pl.BlockSpec((2, _BN, _HALF), lambda i: (0, i, 0)),
        out_shape=jax.ShapeDtypeStruct((2, _N, _HALF), F32),
    )(x, W_in, b_in)


def _tc_layer(hc, msg, dinv, sig, wl, bl, gl, bbl):
    return pl.pallas_call(
        _tc_layer_body,
        grid=(_GRID,),
        in_specs=[
            pl.BlockSpec((2, _BN, _HALF), lambda i: (0, i, 0)),
            pl.BlockSpec((2, _BN, _HALF), lambda i: (0, i, 0)),
            pl.BlockSpec((_BN, 128), lambda i: (i, 0)),
            pl.BlockSpec((_BN, _CHUNK), lambda i: (i, 0)),
            pl.BlockSpec((2 * _HID, _CHUNK), lambda i: (0, 0)),
            pl.BlockSpec((1, _CHUNK), lambda i: (0, 0)),
            pl.BlockSpec((1, _HID), lambda i: (0, 0)),
            pl.BlockSpec((1, _HID), lambda i: (0, 0)),
        ],
        out_specs=[
            pl.BlockSpec((2, _BN, _HALF), lambda i: (0, i, 0)),
            pl.BlockSpec((_BN, _CHUNK), lambda i: (i, 0)),
        ],
        out_shape=[
            jax.ShapeDtypeStruct((2, _N, _HALF), F32),
            jax.ShapeDtypeStruct((_N, _CHUNK), F32),
        ],
    )(hc, msg, dinv, sig, wl, bl, gl, bbl)


def _tc_output(hc, W_out, b_out):
    return pl.pallas_call(
        _tc_out_body,
        grid=(_GRID,),
        in_specs=[
            pl.BlockSpec((2, _BN, _HALF), lambda i: (0, i, 0)),
            pl.BlockSpec((_HID, _OUT), lambda i: (0, 0)),
            pl.BlockSpec((1, _OUT), lambda i: (0, 0)),
        ],
        out_specs=pl.BlockSpec((_BN, _OUT), lambda i: (i, 0)),
        out_shape=jax.ShapeDtypeStruct((_N, _OUT), F32),
    )(hc, W_out, b_out)


def kernel(x, edge_index, W_in, b_in, tm_W, tm_b, ln_g, ln_b, W_out, b_out):
    pad = _EP - _E
    srcp = jnp.concatenate([edge_index[0], jnp.zeros((pad,), jnp.int32)])
    dstp = jnp.concatenate([edge_index[1], jnp.full((pad,), _N, jnp.int32)])
    src3 = srcp.reshape(_IDXROWS, 128)
    dst3 = dstp.reshape(_IDXROWS, 128)

    hc = _tc_input(x, W_in, b_in.reshape(1, _HID))
    dinv = _sc_deg(dst3)
    sig = jnp.zeros((_N, _CHUNK), F32)
    for l in range(_L):
        msg = _sc_agg(hc.reshape(2 * _N, _HALF), src3, dst3)
        hc, sig = _tc_layer(hc, msg, dinv, sig, tm_W[l],
                            tm_b[l].reshape(1, _CHUNK),
                            ln_g[l].reshape(1, _HID), ln_b[l].reshape(1, _HID))
    return _tc_output(hc, W_out, b_out.reshape(1, _OUT))
